# Initial kernel scaffold; baseline (speedup 1.0000x reference)
import jax
import jax.numpy as jnp
from jax.experimental import pallas as pl


def kernel(x, thresholds, weights, ordinals):
    raise NotImplementedError("write your pallas kernel here")


# speedup vs baseline: 185.7829x; 185.7829x over previous
import functools

import jax
import jax.numpy as jnp
from jax import lax
from jax.experimental import pallas as pl
from jax.experimental.pallas import tpu as pltpu
from jax.experimental.pallas import tpu_sc as plsc

B = 8192
IN = 768
OUT = 512
DEPTH = 8
NLEAF = 1 << DEPTH

NC = 2
NS = 16
NW = NC * NS
OPW = OUT // NW
L = 16
CB = 256
NB = B // CB
ROWS = OPW * DEPTH
HCB = CB // 2


def _fern_body(xg, thb, wf, ordf, out,
               rows0, rows1, idx0, idx1, ob0, ob1, ordv, thv, wv,
               gs0, gs1, os0, os1):
    wid = lax.axis_index("s") * NC + lax.axis_index("c")

    rows_ = (rows0, rows1)
    idx_ = (idx0, idx1)
    ob_ = (ob0, ob1)
    gs_ = (gs0, gs1)
    os_ = (os0, os1)

    pltpu.sync_copy(ordf.at[pl.ds(wid * ROWS, ROWS)], ordv)
    pltpu.sync_copy(thb.at[pl.ds(wid * ROWS * L, ROWS * L)], thv)
    pltpu.sync_copy(wf.at[pl.ds(wid * OPW * NLEAF, OPW * NLEAF)], wv)

    iota = lax.iota(jnp.int32, L)

    def start_gather(pb, c):
        off = jnp.full((L,), c * IN, jnp.int32)
        for k in range(ROWS // L):
            sl = pl.ds(k * L, L)
            idx_[pb][sl] = ordv[sl] + off
        pltpu.async_copy(xg.at[idx_[pb]], rows_[pb], gs_[pb])

    def wait_gather(pb):
        pltpu.make_async_copy(xg.at[idx_[pb]], rows_[pb], gs_[pb]).wait()

    def out_slice(c):
        return out.at[pl.ds(wid * OPW, OPW), pl.ds(c * CB, CB)]

    def start_out(pb, c):
        pltpu.async_copy(ob_[pb], out_slice(c), os_[pb])

    def wait_out(pb, c):
        pltpu.make_async_copy(ob_[pb], out_slice(c), os_[pb]).wait()

    def compute(pb):
        rows = rows_[pb]
        ob = ob_[pb]

        def bv_body(bv, carry):
            base = bv * L
            for ol in range(OPW):
                key = jnp.zeros((L,), jnp.int32)
                mm = jnp.full((L,), jnp.inf, jnp.float32)
                for d in range(DEPTH):
                    r = ol * DEPTH + d
                    m = rows[r, pl.ds(base, L)] - thv[pl.ds(r * L, L)]
                    mm = jnp.minimum(mm, jnp.abs(m))
                    key = key | jnp.where(m > 0, jnp.int32(1 << d),
                                          jnp.int32(0))
                wsel = plsc.load_gather(wv, [key + jnp.int32(ol * NLEAF)])
                ob[ol, pl.ds(base, L)] = mm * wsel
            return carry

        lax.fori_loop(0, CB // L, bv_body, 0)

    start_gather(0, 0)

    def step(i, carry):
        for pb in (0, 1):
            c = 2 * i + pb

            @pl.when(c + 1 < NB)
            def _():
                start_gather(1 - pb, c + 1)

            wait_gather(pb)

            @pl.when(c >= 2)
            def _():
                wait_out(pb, c - 2)

            compute(pb)
            start_out(pb, c)
        return carry

    lax.fori_loop(0, NB // 2, step, 0)
    wait_out(0, NB - 2)
    wait_out(1, NB - 1)


_fern_call = functools.partial(
    pl.kernel,
    mesh=plsc.VectorSubcoreMesh(core_axis_name="c", subcore_axis_name="s"),
    compiler_params=pltpu.CompilerParams(needs_layout_passes=False),
    out_type=jax.ShapeDtypeStruct((OUT, B), jnp.float32),
    scratch_types=[
        pltpu.VMEM((ROWS, CB), jnp.float32),
        pltpu.VMEM((ROWS, CB), jnp.float32),
        pltpu.VMEM((ROWS,), jnp.int32),
        pltpu.VMEM((ROWS,), jnp.int32),
        pltpu.VMEM((OPW, CB), jnp.float32),
        pltpu.VMEM((OPW, CB), jnp.float32),
        pltpu.VMEM((ROWS,), jnp.int32),
        pltpu.VMEM((ROWS * L,), jnp.float32),
        pltpu.VMEM((OPW * NLEAF,), jnp.float32),
        pltpu.SemaphoreType.DMA,
        pltpu.SemaphoreType.DMA,
        pltpu.SemaphoreType.DMA,
        pltpu.SemaphoreType.DMA,
    ],
)(_fern_body)


def kernel(x, thresholds, weights, ordinals):
    xg = x.T.reshape(IN, NB, CB).transpose(1, 0, 2).reshape(NB * IN, CB)
    thb = jnp.broadcast_to(thresholds.reshape(OUT * DEPTH, 1),
                           (OUT * DEPTH, L)).reshape(OUT * DEPTH * L)
    wf = weights.reshape(OUT * NLEAF)
    ordf = ordinals.reshape(OUT * DEPTH)
    return _fern_call(xg, thb, wf, ordf).T


# speedup vs baseline: 223.9783x; 1.2056x over previous
import functools

import jax
import jax.numpy as jnp
from jax import lax
from jax.experimental import pallas as pl
from jax.experimental.pallas import tpu as pltpu
from jax.experimental.pallas import tpu_sc as plsc

B = 8192
IN = 768
OUT = 512
DEPTH = 8
NLEAF = 1 << DEPTH

NC = 2
NS = 16
NW = NC * NS
OPW = OUT // NW
L = 16
CB = 256
NB = B // CB
ROWS = OPW * DEPTH
HCB = CB // 2


def _fern_body(xg, thb, wf, ordf, out,
               rows0, rows1, idx0, idx1, ob0, ob1, ordv, thv, wv,
               gs0, gs1, os0, os1):
    wid = lax.axis_index("s") * NC + lax.axis_index("c")

    rows_ = (rows0, rows1)
    idx_ = (idx0, idx1)
    ob_ = (ob0, ob1)
    gs_ = (gs0, gs1)
    os_ = (os0, os1)

    pltpu.sync_copy(ordf.at[pl.ds(wid * ROWS, ROWS)], ordv)
    pltpu.sync_copy(thb.at[pl.ds(wid * ROWS * L, ROWS * L)], thv)
    pltpu.sync_copy(wf.at[pl.ds(wid * OPW * NLEAF, OPW * NLEAF)], wv)

    iota = lax.iota(jnp.int32, L)

    def start_gather(pb, c):
        off = jnp.full((L,), c * IN, jnp.int32)
        for k in range(ROWS // L):
            sl = pl.ds(k * L, L)
            idx_[pb][sl] = ordv[sl] + off
        pltpu.async_copy(xg.at[idx_[pb]], rows_[pb], gs_[pb])

    def wait_gather(pb):
        pltpu.make_async_copy(xg.at[idx_[pb]], rows_[pb], gs_[pb]).wait()

    def out_slice(c):
        return out.at[pl.ds(wid * OPW, OPW), pl.ds(c * CB, CB)]

    def start_out(pb, c):
        pltpu.async_copy(ob_[pb], out_slice(c), os_[pb])

    def wait_out(pb, c):
        pltpu.make_async_copy(ob_[pb], out_slice(c), os_[pb]).wait()

    def compute(pb):
        rows = rows_[pb]
        ob = ob_[pb]

        @plsc.parallel_loop(0, CB, step=L, unroll=2)
        def bv_body(base):
            for ol in range(OPW):
                am = []
                kb = []
                for d in range(DEPTH):
                    r = ol * DEPTH + d
                    m = rows[r, pl.ds(base, L)] - thv[pl.ds(r * L, L)]
                    am.append(jnp.abs(m))
                    kb.append(jnp.where(m > 0, jnp.int32(1 << d),
                                        jnp.int32(0)))
                while len(am) > 1:
                    am = [jnp.minimum(am[i], am[i + 1])
                          for i in range(0, len(am), 2)]
                    kb = [kb[i] | kb[i + 1] for i in range(0, len(kb), 2)]
                wsel = plsc.load_gather(wv, [kb[0] + jnp.int32(ol * NLEAF)])
                ob[ol, pl.ds(base, L)] = am[0] * wsel

    start_gather(0, 0)

    def step(i, carry):
        for pb in (0, 1):
            c = 2 * i + pb

            @pl.when(c + 1 < NB)
            def _():
                start_gather(1 - pb, c + 1)

            wait_gather(pb)

            @pl.when(c >= 2)
            def _():
                wait_out(pb, c - 2)

            compute(pb)
            start_out(pb, c)
        return carry

    lax.fori_loop(0, NB // 2, step, 0)
    wait_out(0, NB - 2)
    wait_out(1, NB - 1)


_fern_call = functools.partial(
    pl.kernel,
    mesh=plsc.VectorSubcoreMesh(core_axis_name="c", subcore_axis_name="s"),
    compiler_params=pltpu.CompilerParams(needs_layout_passes=False),
    out_type=jax.ShapeDtypeStruct((OUT, B), jnp.float32),
    scratch_types=[
        pltpu.VMEM((ROWS, CB), jnp.float32),
        pltpu.VMEM((ROWS, CB), jnp.float32),
        pltpu.VMEM((ROWS,), jnp.int32),
        pltpu.VMEM((ROWS,), jnp.int32),
        pltpu.VMEM((OPW, CB), jnp.float32),
        pltpu.VMEM((OPW, CB), jnp.float32),
        pltpu.VMEM((ROWS,), jnp.int32),
        pltpu.VMEM((ROWS * L,), jnp.float32),
        pltpu.VMEM((OPW * NLEAF,), jnp.float32),
        pltpu.SemaphoreType.DMA,
        pltpu.SemaphoreType.DMA,
        pltpu.SemaphoreType.DMA,
        pltpu.SemaphoreType.DMA,
    ],
)(_fern_body)


def kernel(x, thresholds, weights, ordinals):
    xg = x.T.reshape(IN, NB, CB).transpose(1, 0, 2).reshape(NB * IN, CB)
    thb = jnp.broadcast_to(thresholds.reshape(OUT * DEPTH, 1),
                           (OUT * DEPTH, L)).reshape(OUT * DEPTH * L)
    wf = weights.reshape(OUT * NLEAF)
    ordf = ordinals.reshape(OUT * DEPTH)
    return _fern_call(xg, thb, wf, ordf).T
